# baseline (speedup 1.0000x reference)
import functools
import jax
import jax.numpy as jnp
from jax import lax
from jax.experimental import pallas as pl
from jax.experimental.pallas import tpu as pltpu
from jax.experimental.pallas import tpu_sc as plsc

N = 10000
E = 320000
F = 128
H = 4
HF = 512
D_EDGE = 16
EHAT = E + N
EP = 335872
KCH = 64
TPW_A = EP // 32
TPW_E = EP // 16
NCH_E = TPW_E // KCH
SEG_C = 256
NSEG_C = TPW_A // SEG_C
SEG_E = 512
NSEG_E = TPW_E // SEG_E
NB = 10240
EBLK = 512
NBLK = 400

_mesh = plsc.VectorSubcoreMesh(core_axis_name="c", subcore_axis_name="s")


def _k1_body(x_ref, eW_ref, eb_ref, elnw_ref, elnb_ref, Wl_ref, bl_ref,
             Wr_ref, br_ref, h0_ref, xl_ref, xr_ref, xlb_ref, xrb_ref):
    x = x_ref[...]
    h = jnp.dot(x, eW_ref[...], preferred_element_type=jnp.float32) + eb_ref[...]
    m = jnp.mean(h, axis=-1, keepdims=True)
    v = jnp.mean((h - m) ** 2, axis=-1, keepdims=True)
    h = (h - m) * lax.rsqrt(v + 1e-5) * elnw_ref[...] + elnb_ref[...]
    h0 = jnp.maximum(h, 0.0)
    h0_ref[...] = h0
    xl = jnp.dot(h0, Wl_ref[...], preferred_element_type=jnp.float32) + bl_ref[...]
    xr = jnp.dot(h0, Wr_ref[...], preferred_element_type=jnp.float32) + br_ref[...]
    xl_ref[...] = xl
    xr_ref[...] = xr
    def tobf(xf):
        u = lax.bitcast_convert_type(xf, jnp.uint32)
        return (u + jnp.uint32(0x8000)) >> 16

    def pack(r):
        w = (r[:, HF // 2:] << 16) | r[:, : HF // 2]
        return lax.bitcast_convert_type(w, jnp.int32)

    xlb_ref[...] = pack(tobf(xl))
    xrb_ref[...] = pack(tobf(xr))


def _pre(x, embed_W, embed_b, embed_ln_w, embed_ln_b, Wl, bl, Wr, br):
    row = lambda i: (i, 0)
    fixed = lambda i: (0, 0)
    return pl.pallas_call(
        _k1_body,
        grid=(N // NBLK,),
        in_specs=[
            pl.BlockSpec((NBLK, F), row),
            pl.BlockSpec((F, F), fixed),
            pl.BlockSpec((1, F), fixed),
            pl.BlockSpec((1, F), fixed),
            pl.BlockSpec((1, F), fixed),
            pl.BlockSpec((F, HF), fixed),
            pl.BlockSpec((1, HF), fixed),
            pl.BlockSpec((F, HF), fixed),
            pl.BlockSpec((1, HF), fixed),
        ],
        out_specs=[
            pl.BlockSpec((NBLK, F), row),
            pl.BlockSpec((NBLK, HF), row),
            pl.BlockSpec((NBLK, HF), row),
            pl.BlockSpec((NBLK, HF // 2), lambda i: (i, 0)),
            pl.BlockSpec((NBLK, HF // 2), lambda i: (i, 0)),
        ],
        out_shape=[
            jax.ShapeDtypeStruct((N, F), jnp.float32),
            jax.ShapeDtypeStruct((N, HF), jnp.float32),
            jax.ShapeDtypeStruct((N, HF), jnp.float32),
            jax.ShapeDtypeStruct((N, HF // 2), jnp.int32),
            jax.ShapeDtypeStruct((N, HF // 2), jnp.int32),
        ],
    )(x, embed_W, embed_b.reshape(1, F), embed_ln_w.reshape(1, F),
      embed_ln_b.reshape(1, F), Wl, bl.reshape(1, HF), Wr, br.reshape(1, HF))


KA = 64
NCHA = TPW_A // KA
HW = HF // 2

@functools.partial(
    pl.kernel, mesh=_mesh,
    compiler_params=pltpu.CompilerParams(needs_layout_passes=False),
    out_type=jax.ShapeDtypeStruct((EP, HW), jnp.int32),
    scratch_types=[
        pltpu.VMEM((TPW_A,), jnp.int32),
        pltpu.VMEM((TPW_A,), jnp.int32),
        pltpu.VMEM((KA, HW), jnp.int32),
        pltpu.VMEM((KA, HW), jnp.int32),
        pltpu.SemaphoreType.DMA,
        pltpu.SemaphoreType.DMA,
    ],
)
def _sc_gsum(xlb_hbm, xrb_hbm, src_hbm, dst_hbm, out_hbm,
             src_v, dst_v, bufL, bufR, sem1, sem2):
    wid = lax.axis_index("s") * 2 + lax.axis_index("c")
    base = wid * TPW_A
    pltpu.sync_copy(src_hbm.at[pl.ds(base, TPW_A)], src_v)
    pltpu.sync_copy(dst_hbm.at[pl.ds(base, TPW_A)], dst_v)

    def chunk(i, carry):
        cpL = pltpu.async_copy(
            xlb_hbm.at[src_v.at[pl.ds(i * KA, KA)]], bufL, sem1)
        cpR = pltpu.async_copy(
            xrb_hbm.at[dst_v.at[pl.ds(i * KA, KA)]], bufR, sem2)
        cpL.wait()
        cpR.wait()

        def addrow(k, c2):
            for q in range(HW // 16):
                sq = pl.ds(q * 16, 16)
                a = plsc.bitcast(bufL[k, sq], jnp.bfloat16)
                b = plsc.bitcast(bufR[k, sq], jnp.bfloat16)
                bufL[k, sq] = plsc.bitcast(a + b, jnp.int32)
            return c2
        lax.fori_loop(0, KA, addrow, 0)
        pltpu.sync_copy(bufL, out_hbm.at[pl.ds(base + i * KA, KA), :])
        return carry

    lax.fori_loop(0, NCHA, chunk, 0)


def _k3_body(gs_ref, ea_ref, We_ref, att8_ref, ex_ref):
    i = pl.program_id(0)
    w = lax.bitcast_convert_type(gs_ref[...], jnp.uint32)
    f_lo = lax.bitcast_convert_type(w << 16, jnp.float32)
    f_hi = lax.bitcast_convert_type(w & jnp.uint32(0xFFFF0000), jnp.float32)
    gs = jnp.concatenate([f_lo, f_hi], axis=1)
    z = gs + jnp.dot(ea_ref[...], We_ref[...],
                     preferred_element_type=jnp.float32)
    alphaT = (lax.dot_general(att8_ref[...], z, (((1,), (1,)), ((), ())),
                              preferred_element_type=jnp.float32) * 0.6
              + lax.dot_general(att8_ref[...], jnp.abs(z),
                                (((1,), (1,)), ((), ())),
                                preferred_element_type=jnp.float32) * 0.4)
    pos = i * EBLK + lax.broadcasted_iota(jnp.int32, (8, EBLK), 1)
    hrow = lax.broadcasted_iota(jnp.int32, (8, EBLK), 0)
    valid = (pos < EHAT) & (hrow < H)
    ex_ref[0] = jnp.where(valid, jnp.exp(alphaT), 0.0)


def _edge_alpha(gsum, eap, We, att8):
    fixed = lambda i: (0, 0)
    return pl.pallas_call(
        _k3_body,
        grid=(EP // EBLK,),
        in_specs=[
            pl.BlockSpec((EBLK, HF // 2), lambda i: (i, 0)),
            pl.BlockSpec((EBLK, D_EDGE), lambda i: (i, 0)),
            pl.BlockSpec((D_EDGE, HF), fixed),
            pl.BlockSpec((8, HF), fixed),
        ],
        out_specs=pl.BlockSpec((1, 8, EBLK), lambda i: (0, 0, i)),
        out_shape=jax.ShapeDtypeStruct((1, 8, EP), jnp.float32),
    )(gsum, eap, We, att8)


@functools.partial(
    pl.kernel, mesh=_mesh,
    compiler_params=pltpu.CompilerParams(needs_layout_passes=False),
    out_type=jax.ShapeDtypeStruct((2, 1, H * NB), jnp.float32),
    scratch_types=[
        pltpu.VMEM((H * NB,), jnp.float32),
        pltpu.VMEM((TPW_A,), jnp.int32),
        pltpu.VMEM((H, SEG_C), jnp.float32),
        pltpu.VMEM((H * NB // 16,), jnp.float32),
        pltpu.VMEM((H * NB // 16,), jnp.float32),
        pltpu.VMEM_SHARED((16, H * NB), jnp.float32),
        pltpu.SemaphoreType.DMA,
    ],
)
def _sc_den(dst_hbm, ex_hbm, den_hbm, bins, dst_v, exs, acc, tmp, shared, sem):
    cid = lax.axis_index("c")
    sid = lax.axis_index("s")
    wid = sid * 2 + cid
    base = wid * TPW_A
    pltpu.sync_copy(dst_hbm.at[pl.ds(base, TPW_A)], dst_v)

    zeros16 = jnp.zeros((16,), jnp.float32)

    def zero(i, c):
        bins[pl.ds(i * 16, 16)] = zeros16
        return c
    lax.fori_loop(0, H * NB // 16, zero, 0)

    def seg(t, c):
        for h in range(H):
            pltpu.sync_copy(ex_hbm.at[0, h, pl.ds(base + t * SEG_C, SEG_C)],
                            exs.at[h])

        def grp(g, c2):
            idx = dst_v[pl.ds(t * SEG_C + g * 16, 16)]
            for h in range(H):
                vals = exs[h, pl.ds(g * 16, 16)]
                plsc.addupdate_scatter(bins, [idx + (h * NB)], vals)
            return c2
        lax.fori_loop(0, SEG_C // 16, grp, 0)
        return c
    lax.fori_loop(0, NSEG_C, seg, 0)

    pltpu.sync_copy(bins, shared.at[sid])
    plsc.subcore_barrier()
    W = H * NB // 16
    pltpu.sync_copy(shared.at[0, pl.ds(sid * W, W)], acc)
    for p in range(1, 16):
        pltpu.sync_copy(shared.at[p, pl.ds(sid * W, W)], tmp)

        def addv(i, c):
            s_ = pl.ds(i * 16, 16)
            acc[s_] = acc[s_] + tmp[s_]
            return c
        lax.fori_loop(0, W // 16, addv, 0)
    pltpu.sync_copy(acc, den_hbm.at[cid, 0, pl.ds(sid * W, W)])


N2 = 10240
CPS = SEG_E // KCH
NPAIR_E = CPS // 2

@functools.partial(
    pl.kernel, mesh=_mesh,
    compiler_params=pltpu.CompilerParams(needs_layout_passes=False),
    out_type=jax.ShapeDtypeStruct((H, N2, F), jnp.float32),
    scratch_types=[
        pltpu.VMEM((CPS, KCH), jnp.int32),
        pltpu.VMEM((CPS, KCH), jnp.int32),
        pltpu.VMEM((SEG_E,), jnp.float32),
        pltpu.VMEM((NB,), jnp.float32),
        pltpu.VMEM((NB,), jnp.float32),
        pltpu.VMEM((KCH, F), jnp.float32),
        pltpu.VMEM((KCH, F), jnp.float32),
        pltpu.VMEM_SHARED((N2, F), jnp.float32),
        pltpu.SemaphoreType.DMA,
        pltpu.SemaphoreType.DMA,
        pltpu.SemaphoreType.DMA,
        pltpu.SemaphoreType.DMA,
    ],
)
def _sc_msg(xlT_hbm, src2_hbm, dst2_hbm, ex_hbm, den_hbm, out_hbm,
            src_v, dst_v, wts, den_v, dtmp, rows0, rows1, bins,
            sg0, sg1, ss0, ss1):
    cid = lax.axis_index("c")
    sid = lax.axis_index("s")

    rowbase = sid * NCH_E
    exbase = sid * TPW_E

    zeros16 = jnp.zeros((16,), jnp.float32)

    for p in range(2):
        h = cid * 2 + p
        pltpu.sync_copy(den_hbm.at[0, 0, pl.ds(h * NB, NB)], den_v)
        pltpu.sync_copy(den_hbm.at[1, 0, pl.ds(h * NB, NB)], dtmp)

        def addden(i, c):
            s_ = pl.ds(i * 16, 16)
            den_v[s_] = den_v[s_] + dtmp[s_] + 1e-16
            return c
        lax.fori_loop(0, NB // 16, addden, 0)

        def zrow(i, c):
            for j in range(F // 16):
                rows0[i, pl.ds(j * 16, 16)] = zeros16
            return c
        lax.fori_loop(0, KCH, zrow, 0)
        for q in range(10):
            pltpu.sync_copy(rows0, bins.at[pl.ds(sid * 640 + q * 64, 64), :])
        plsc.subcore_barrier()

        def gath(i, rows, sg):
            pltpu.async_copy(xlT_hbm.at[h].at[src_v.at[i]], rows, sg)

        def consume(i, rows, sg, ss):
            pltpu.make_async_copy(
                xlT_hbm.at[h].at[pl.ds(0, KCH), :], rows, sg).wait()

            def scale(k, c):
                w = plsc.load_gather(wts, [jnp.full((16,), i * KCH + k,
                                                    jnp.int32)])
                for j in range(F // 16):
                    s_ = pl.ds(j * 16, 16)
                    rows[k, s_] = rows[k, s_] * w
                return c
            lax.fori_loop(0, KCH, scale, 0)
            pltpu.async_copy(rows, bins.at[dst_v.at[i]], ss, add=True)

        def waits(rows, ss):
            pltpu.make_async_copy(rows, bins.at[pl.ds(0, KCH), :], ss).wait()

        def seg(t, carry):
            pltpu.sync_copy(
                src2_hbm.at[pl.ds(rowbase + t * CPS, CPS), :], src_v)
            pltpu.sync_copy(
                dst2_hbm.at[pl.ds(rowbase + t * CPS, CPS), :], dst_v)
            pltpu.sync_copy(
                ex_hbm.at[0, h, pl.ds(exbase + t * SEG_E, SEG_E)], wts)

            def wcalc(g, c):
                s_ = pl.ds(g * 16, 16)
                idx = dst_v[g // (KCH // 16), pl.ds((g % (KCH // 16)) * 16, 16)]
                denv = plsc.load_gather(den_v, [idx])
                wts[s_] = wts[s_] / denv
                return c
            lax.fori_loop(0, SEG_E // 16, wcalc, 0)

            gath(0, rows0, sg0)

            def pair(u, carry2):
                @pl.when(u > 0)
                def _():
                    waits(rows1, ss1)
                gath(2 * u + 1, rows1, sg1)
                consume(2 * u, rows0, sg0, ss0)
                consume(2 * u + 1, rows1, sg1, ss1)
                waits(rows0, ss0)

                @pl.when(u < NPAIR_E - 1)
                def _():
                    gath(2 * u + 2, rows0, sg0)
                return carry2

            lax.fori_loop(0, NPAIR_E, pair, 0)
            waits(rows1, ss1)
            return carry

        lax.fori_loop(0, NSEG_E, seg, 0)
        plsc.subcore_barrier()
        for q in range(10):
            r0 = sid * 640 + q * 64
            pltpu.sync_copy(bins.at[pl.ds(r0, 64), :],
                            out_hbm.at[h].at[pl.ds(r0, 64), :])
        plsc.subcore_barrier()


def _k6a_body(o_ref, b_ref, acc_ref):
    i = pl.program_id(0)
    o = o_ref[...] + b_ref[...]

    @pl.when(i == 0)
    def _init():
        acc_ref[0] = 0.0
        acc_ref[1] = 0.0
    acc_ref[0] += jnp.sum(o)
    acc_ref[1] += jnp.sum(o * o)


def _k6b_body(o_ref, b_ref, stats_ref, glnw_ref, glnb_ref, pW_ref, pb_ref,
              plnw_ref, plnb_ref, h0_ref, out_ref):
    cnt = float(N * HF)
    mu = stats_ref[0] / cnt
    ms2 = stats_ref[1] / cnt
    sd = jnp.sqrt(jnp.maximum(ms2 - mu * mu, 0.0)) + 1e-5
    o = (o_ref[...] + b_ref[...] - mu) / sd * glnw_ref[...] + glnb_ref[...]
    o = jnp.maximum(o, 0.0)
    h = jnp.dot(o, pW_ref[...], preferred_element_type=jnp.float32) + pb_ref[...]
    m = jnp.mean(h, axis=-1, keepdims=True)
    v = jnp.mean((h - m) ** 2, axis=-1, keepdims=True)
    h = (h - m) * lax.rsqrt(v + 1e-5) * plnw_ref[...] + plnb_ref[...]
    out_ref[...] = jnp.maximum(h0_ref[...], jnp.maximum(h, 0.0))


def _post(out512, gat_bias, gln_w, gln_b, postW, postb, post_ln_w, post_ln_b, h0):
    fixed = lambda i: (0, 0)
    stats = pl.pallas_call(
        _k6a_body,
        grid=(N // NBLK,),
        in_specs=[pl.BlockSpec((NBLK, HF), lambda i: (i, 0)),
                  pl.BlockSpec((1, HF), fixed)],
        out_specs=pl.BlockSpec(memory_space=pltpu.SMEM),
        out_shape=jax.ShapeDtypeStruct((2,), jnp.float32),
    )(out512, gat_bias.reshape(1, HF))
    return pl.pallas_call(
        _k6b_body,
        grid=(N // NBLK,),
        in_specs=[
            pl.BlockSpec((NBLK, HF), lambda i: (i, 0)),
            pl.BlockSpec((1, HF), fixed),
            pl.BlockSpec(memory_space=pltpu.SMEM),
            pl.BlockSpec((1, HF), fixed),
            pl.BlockSpec((1, HF), fixed),
            pl.BlockSpec((HF, F), fixed),
            pl.BlockSpec((1, F), fixed),
            pl.BlockSpec((1, F), fixed),
            pl.BlockSpec((1, F), fixed),
            pl.BlockSpec((NBLK, F), lambda i: (i, 0)),
        ],
        out_specs=pl.BlockSpec((NBLK, F), lambda i: (i, 0)),
        out_shape=jax.ShapeDtypeStruct((N, F), jnp.float32),
    )(out512, gat_bias.reshape(1, HF), stats, gln_w.reshape(1, HF),
      gln_b.reshape(1, HF), postW, postb.reshape(1, F),
      post_ln_w.reshape(1, F), post_ln_b.reshape(1, F), h0)


def kernel(x, edge_index, edge_attr, embed_W, embed_b, embed_ln_w, embed_ln_b,
           Wl, bl, Wr, br, We, att, gat_bias, gln_w, gln_b,
           postW, postb, post_ln_w, post_ln_b):
    h0, xl, xr, xlb, xrb = _pre(x, embed_W, embed_b, embed_ln_w, embed_ln_b,
                                Wl, bl, Wr, br)

    loop = jnp.arange(N, dtype=jnp.int32)
    pad = jnp.zeros((EP - EHAT,), jnp.int32)
    srcp = jnp.concatenate([edge_index[0].astype(jnp.int32), loop, pad])
    dstp = jnp.concatenate([edge_index[1].astype(jnp.int32), loop, pad])
    eap = jnp.concatenate(
        [edge_attr, jnp.zeros((EP - E, D_EDGE), edge_attr.dtype)], axis=0)

    gsum = _sc_gsum(xlb, xrb, srcp, dstp)

    att_flat = att.reshape(1, HF)
    hsel = (jnp.arange(HF) // F)[None, :] == jnp.arange(8)[:, None]
    att8 = jnp.where(hsel, att_flat, 0.0).astype(jnp.float32)
    exT = _edge_alpha(gsum, eap, We, att8)

    den2 = _sc_den(dstp, exT)

    xlT = xl.reshape(N, H, F).transpose(1, 0, 2)
    src2 = srcp.reshape(EP // KCH, KCH)
    dst2 = dstp.reshape(EP // KCH, KCH)
    outT = _sc_msg(xlT, src2, dst2, exT, den2)

    out512 = outT[:, :N, :].transpose(1, 0, 2).reshape(N, HF)
    return _post(out512, gat_bias, gln_w, gln_b, postW, postb,
                 post_ln_w, post_ln_b, h0)


# speedup vs baseline: 1.0132x; 1.0132x over previous
import functools
import jax
import jax.numpy as jnp
from jax import lax
from jax.experimental import pallas as pl
from jax.experimental.pallas import tpu as pltpu
from jax.experimental.pallas import tpu_sc as plsc

N = 10000
E = 320000
F = 128
H = 4
HF = 512
D_EDGE = 16
EHAT = E + N
EP = 335872
KCH = 64
TPW_A = EP // 32
TPW_E = EP // 16
NCH_E = TPW_E // KCH
SEG_C = 256
NSEG_C = TPW_A // SEG_C
SEG_E = 512
NSEG_E = TPW_E // SEG_E
NB = 10240
EBLK = 512
NBLK = 400

_mesh = plsc.VectorSubcoreMesh(core_axis_name="c", subcore_axis_name="s")


def _k1_body(x_ref, eW_ref, eb_ref, elnw_ref, elnb_ref, Wl_ref, bl_ref,
             Wr_ref, br_ref, h0_ref, xl_ref, xr_ref, xlb_ref, xrb_ref):
    x = x_ref[...]
    h = jnp.dot(x, eW_ref[...], preferred_element_type=jnp.float32) + eb_ref[...]
    m = jnp.mean(h, axis=-1, keepdims=True)
    v = jnp.mean((h - m) ** 2, axis=-1, keepdims=True)
    h = (h - m) * lax.rsqrt(v + 1e-5) * elnw_ref[...] + elnb_ref[...]
    h0 = jnp.maximum(h, 0.0)
    h0_ref[...] = h0
    xl = jnp.dot(h0, Wl_ref[...], preferred_element_type=jnp.float32) + bl_ref[...]
    xr = jnp.dot(h0, Wr_ref[...], preferred_element_type=jnp.float32) + br_ref[...]
    xl_ref[...] = xl
    xr_ref[...] = xr
    def tobf(xf):
        u = lax.bitcast_convert_type(xf, jnp.uint32)
        return (u + jnp.uint32(0x8000)) >> 16

    def pack(r):
        w = (r[:, HF // 2:] << 16) | r[:, : HF // 2]
        return lax.bitcast_convert_type(w, jnp.int32)

    xlb_ref[...] = pack(tobf(xl))
    xrb_ref[...] = pack(tobf(xr))


def _pre(x, embed_W, embed_b, embed_ln_w, embed_ln_b, Wl, bl, Wr, br):
    row = lambda i: (i, 0)
    fixed = lambda i: (0, 0)
    return pl.pallas_call(
        _k1_body,
        grid=(N // NBLK,),
        in_specs=[
            pl.BlockSpec((NBLK, F), row),
            pl.BlockSpec((F, F), fixed),
            pl.BlockSpec((1, F), fixed),
            pl.BlockSpec((1, F), fixed),
            pl.BlockSpec((1, F), fixed),
            pl.BlockSpec((F, HF), fixed),
            pl.BlockSpec((1, HF), fixed),
            pl.BlockSpec((F, HF), fixed),
            pl.BlockSpec((1, HF), fixed),
        ],
        out_specs=[
            pl.BlockSpec((NBLK, F), row),
            pl.BlockSpec((NBLK, HF), row),
            pl.BlockSpec((NBLK, HF), row),
            pl.BlockSpec((NBLK, HF // 2), lambda i: (i, 0)),
            pl.BlockSpec((NBLK, HF // 2), lambda i: (i, 0)),
        ],
        out_shape=[
            jax.ShapeDtypeStruct((N, F), jnp.float32),
            jax.ShapeDtypeStruct((N, HF), jnp.float32),
            jax.ShapeDtypeStruct((N, HF), jnp.float32),
            jax.ShapeDtypeStruct((N, HF // 2), jnp.int32),
            jax.ShapeDtypeStruct((N, HF // 2), jnp.int32),
        ],
    )(x, embed_W, embed_b.reshape(1, F), embed_ln_w.reshape(1, F),
      embed_ln_b.reshape(1, F), Wl, bl.reshape(1, HF), Wr, br.reshape(1, HF))


KA = 64
NCHA = TPW_A // KA
HW = HF // 2

@functools.partial(
    pl.kernel, mesh=_mesh,
    compiler_params=pltpu.CompilerParams(needs_layout_passes=False),
    out_type=jax.ShapeDtypeStruct((EP, HW), jnp.int32),
    scratch_types=[
        pltpu.VMEM((TPW_A,), jnp.int32),
        pltpu.VMEM((TPW_A,), jnp.int32),
        pltpu.VMEM((KA, HW), jnp.int32),
        pltpu.VMEM((KA, HW), jnp.int32),
        pltpu.SemaphoreType.DMA,
        pltpu.SemaphoreType.DMA,
    ],
)
def _sc_gsum(xlb_hbm, xrb_hbm, src_hbm, dst_hbm, out_hbm,
             src_v, dst_v, bufL, bufR, sem1, sem2):
    wid = lax.axis_index("s") * 2 + lax.axis_index("c")
    base = wid * TPW_A
    pltpu.sync_copy(src_hbm.at[pl.ds(base, TPW_A)], src_v)
    pltpu.sync_copy(dst_hbm.at[pl.ds(base, TPW_A)], dst_v)

    def chunk(i, carry):
        cpL = pltpu.async_copy(
            xlb_hbm.at[src_v.at[pl.ds(i * KA, KA)]], bufL, sem1)
        cpR = pltpu.async_copy(
            xrb_hbm.at[dst_v.at[pl.ds(i * KA, KA)]], bufR, sem2)
        cpL.wait()
        cpR.wait()

        def addrow(k, c2):
            for q in range(HW // 16):
                sq = pl.ds(q * 16, 16)
                a = plsc.bitcast(bufL[k, sq], jnp.bfloat16)
                b = plsc.bitcast(bufR[k, sq], jnp.bfloat16)
                bufL[k, sq] = plsc.bitcast(a + b, jnp.int32)
            return c2
        lax.fori_loop(0, KA, addrow, 0)
        pltpu.sync_copy(bufL, out_hbm.at[pl.ds(base + i * KA, KA), :])
        return carry

    lax.fori_loop(0, NCHA, chunk, 0)


def _k3_body(gs_ref, ea_ref, We_ref, att8_ref, ex_ref):
    i = pl.program_id(0)
    w = lax.bitcast_convert_type(gs_ref[...], jnp.uint32)
    f_lo = lax.bitcast_convert_type(w << 16, jnp.float32)
    f_hi = lax.bitcast_convert_type(w & jnp.uint32(0xFFFF0000), jnp.float32)
    gs = jnp.concatenate([f_lo, f_hi], axis=1)
    z = gs + jnp.dot(ea_ref[...], We_ref[...],
                     preferred_element_type=jnp.float32)
    zl = jnp.maximum(z, 0.2 * z)
    alphaT = lax.dot_general(att8_ref[...], zl, (((1,), (1,)), ((), ())),
                             preferred_element_type=jnp.float32)
    pos = i * EBLK + lax.broadcasted_iota(jnp.int32, (8, EBLK), 1)
    hrow = lax.broadcasted_iota(jnp.int32, (8, EBLK), 0)
    valid = (pos < EHAT) & (hrow < H)
    ex_ref[0] = jnp.where(valid, jnp.exp(alphaT), 0.0)


def _edge_alpha(gsum, eap, We, att8):
    fixed = lambda i: (0, 0)
    return pl.pallas_call(
        _k3_body,
        grid=(EP // EBLK,),
        in_specs=[
            pl.BlockSpec((EBLK, HF // 2), lambda i: (i, 0)),
            pl.BlockSpec((EBLK, D_EDGE), lambda i: (i, 0)),
            pl.BlockSpec((D_EDGE, HF), fixed),
            pl.BlockSpec((8, HF), fixed),
        ],
        out_specs=pl.BlockSpec((1, 8, EBLK), lambda i: (0, 0, i)),
        out_shape=jax.ShapeDtypeStruct((1, 8, EP), jnp.float32),
    )(gsum, eap, We, att8)


@functools.partial(
    pl.kernel, mesh=_mesh,
    compiler_params=pltpu.CompilerParams(needs_layout_passes=False),
    out_type=jax.ShapeDtypeStruct((2, 1, H * NB), jnp.float32),
    scratch_types=[
        pltpu.VMEM((H * NB,), jnp.float32),
        pltpu.VMEM((TPW_A,), jnp.int32),
        pltpu.VMEM((H, SEG_C), jnp.float32),
        pltpu.VMEM((H * NB // 16,), jnp.float32),
        pltpu.VMEM((H * NB // 16,), jnp.float32),
        pltpu.VMEM_SHARED((16, H * NB), jnp.float32),
        pltpu.SemaphoreType.DMA,
    ],
)
def _sc_den(dst_hbm, ex_hbm, den_hbm, bins, dst_v, exs, acc, tmp, shared, sem):
    cid = lax.axis_index("c")
    sid = lax.axis_index("s")
    wid = sid * 2 + cid
    base = wid * TPW_A
    pltpu.sync_copy(dst_hbm.at[pl.ds(base, TPW_A)], dst_v)

    zeros16 = jnp.zeros((16,), jnp.float32)

    def zero(i, c):
        bins[pl.ds(i * 16, 16)] = zeros16
        return c
    lax.fori_loop(0, H * NB // 16, zero, 0)

    def seg(t, c):
        for h in range(H):
            pltpu.sync_copy(ex_hbm.at[0, h, pl.ds(base + t * SEG_C, SEG_C)],
                            exs.at[h])

        def grp(g, c2):
            idx = dst_v[pl.ds(t * SEG_C + g * 16, 16)]
            for h in range(H):
                vals = exs[h, pl.ds(g * 16, 16)]
                plsc.addupdate_scatter(bins, [idx + (h * NB)], vals)
            return c2
        lax.fori_loop(0, SEG_C // 16, grp, 0)
        return c
    lax.fori_loop(0, NSEG_C, seg, 0)

    pltpu.sync_copy(bins, shared.at[sid])
    plsc.subcore_barrier()
    W = H * NB // 16
    pltpu.sync_copy(shared.at[0, pl.ds(sid * W, W)], acc)
    for p in range(1, 16):
        pltpu.sync_copy(shared.at[p, pl.ds(sid * W, W)], tmp)

        def addv(i, c):
            s_ = pl.ds(i * 16, 16)
            acc[s_] = acc[s_] + tmp[s_]
            return c
        lax.fori_loop(0, W // 16, addv, 0)
    pltpu.sync_copy(acc, den_hbm.at[cid, 0, pl.ds(sid * W, W)])


N2 = 10240
CPS = SEG_E // KCH
NPAIR_E = CPS // 2

@functools.partial(
    pl.kernel, mesh=_mesh,
    compiler_params=pltpu.CompilerParams(needs_layout_passes=False),
    out_type=jax.ShapeDtypeStruct((H, N2, F), jnp.float32),
    scratch_types=[
        pltpu.VMEM((CPS, KCH), jnp.int32),
        pltpu.VMEM((CPS, KCH), jnp.int32),
        pltpu.VMEM((SEG_E,), jnp.float32),
        pltpu.VMEM((NB,), jnp.float32),
        pltpu.VMEM((NB,), jnp.float32),
        pltpu.VMEM((KCH, F), jnp.float32),
        pltpu.VMEM((KCH, F), jnp.float32),
        pltpu.VMEM_SHARED((N2, F), jnp.float32),
        pltpu.SemaphoreType.DMA,
        pltpu.SemaphoreType.DMA,
        pltpu.SemaphoreType.DMA,
        pltpu.SemaphoreType.DMA,
    ],
)
def _sc_msg(xlT_hbm, src2_hbm, dst2_hbm, ex_hbm, den_hbm, out_hbm,
            src_v, dst_v, wts, den_v, dtmp, rows0, rows1, bins,
            sg0, sg1, ss0, ss1):
    cid = lax.axis_index("c")
    sid = lax.axis_index("s")

    rowbase = sid * NCH_E
    exbase = sid * TPW_E

    zeros16 = jnp.zeros((16,), jnp.float32)

    for p in range(2):
        h = cid * 2 + p
        pltpu.sync_copy(den_hbm.at[0, 0, pl.ds(h * NB, NB)], den_v)
        pltpu.sync_copy(den_hbm.at[1, 0, pl.ds(h * NB, NB)], dtmp)

        def addden(i, c):
            s_ = pl.ds(i * 16, 16)
            den_v[s_] = den_v[s_] + dtmp[s_] + 1e-16
            return c
        lax.fori_loop(0, NB // 16, addden, 0)

        def zrow(i, c):
            for j in range(F // 16):
                rows0[i, pl.ds(j * 16, 16)] = zeros16
            return c
        lax.fori_loop(0, KCH, zrow, 0)
        for q in range(10):
            pltpu.sync_copy(rows0, bins.at[pl.ds(sid * 640 + q * 64, 64), :])
        plsc.subcore_barrier()

        def gath(i, rows, sg):
            pltpu.async_copy(xlT_hbm.at[h].at[src_v.at[i]], rows, sg)

        def consume(i, rows, sg, ss):
            pltpu.make_async_copy(
                xlT_hbm.at[h].at[pl.ds(0, KCH), :], rows, sg).wait()

            def scale(k, c):
                w = plsc.load_gather(wts, [jnp.full((16,), i * KCH + k,
                                                    jnp.int32)])
                for j in range(F // 16):
                    s_ = pl.ds(j * 16, 16)
                    rows[k, s_] = rows[k, s_] * w
                return c
            lax.fori_loop(0, KCH, scale, 0)
            pltpu.async_copy(rows, bins.at[dst_v.at[i]], ss, add=True)

        def waits(rows, ss):
            pltpu.make_async_copy(rows, bins.at[pl.ds(0, KCH), :], ss).wait()

        def seg(t, carry):
            pltpu.sync_copy(
                src2_hbm.at[pl.ds(rowbase + t * CPS, CPS), :], src_v)
            pltpu.sync_copy(
                dst2_hbm.at[pl.ds(rowbase + t * CPS, CPS), :], dst_v)
            pltpu.sync_copy(
                ex_hbm.at[0, h, pl.ds(exbase + t * SEG_E, SEG_E)], wts)

            def wcalc(g, c):
                s_ = pl.ds(g * 16, 16)
                idx = dst_v[g // (KCH // 16), pl.ds((g % (KCH // 16)) * 16, 16)]
                denv = plsc.load_gather(den_v, [idx])
                wts[s_] = wts[s_] / denv
                return c
            lax.fori_loop(0, SEG_E // 16, wcalc, 0)

            gath(0, rows0, sg0)

            def pair(u, carry2):
                @pl.when(u > 0)
                def _():
                    waits(rows1, ss1)
                gath(2 * u + 1, rows1, sg1)
                consume(2 * u, rows0, sg0, ss0)
                consume(2 * u + 1, rows1, sg1, ss1)
                waits(rows0, ss0)

                @pl.when(u < NPAIR_E - 1)
                def _():
                    gath(2 * u + 2, rows0, sg0)
                return carry2

            lax.fori_loop(0, NPAIR_E, pair, 0)
            waits(rows1, ss1)
            return carry

        lax.fori_loop(0, NSEG_E, seg, 0)
        plsc.subcore_barrier()
        for q in range(10):
            r0 = sid * 640 + q * 64
            pltpu.sync_copy(bins.at[pl.ds(r0, 64), :],
                            out_hbm.at[h].at[pl.ds(r0, 64), :])
        plsc.subcore_barrier()


def _k6a_body(o_ref, b_ref, acc_ref):
    i = pl.program_id(0)
    o = o_ref[...] + b_ref[...]

    @pl.when(i == 0)
    def _init():
        acc_ref[0] = 0.0
        acc_ref[1] = 0.0
    acc_ref[0] += jnp.sum(o)
    acc_ref[1] += jnp.sum(o * o)


def _k6b_body(o_ref, b_ref, stats_ref, glnw_ref, glnb_ref, pW_ref, pb_ref,
              plnw_ref, plnb_ref, h0_ref, out_ref):
    cnt = float(N * HF)
    mu = stats_ref[0] / cnt
    ms2 = stats_ref[1] / cnt
    sd = jnp.sqrt(jnp.maximum(ms2 - mu * mu, 0.0)) + 1e-5
    o = (o_ref[...] + b_ref[...] - mu) / sd * glnw_ref[...] + glnb_ref[...]
    o = jnp.maximum(o, 0.0)
    h = jnp.dot(o, pW_ref[...], preferred_element_type=jnp.float32) + pb_ref[...]
    m = jnp.mean(h, axis=-1, keepdims=True)
    v = jnp.mean((h - m) ** 2, axis=-1, keepdims=True)
    h = (h - m) * lax.rsqrt(v + 1e-5) * plnw_ref[...] + plnb_ref[...]
    out_ref[...] = jnp.maximum(h0_ref[...], jnp.maximum(h, 0.0))


def _post(out512, gat_bias, gln_w, gln_b, postW, postb, post_ln_w, post_ln_b, h0):
    fixed = lambda i: (0, 0)
    stats = pl.pallas_call(
        _k6a_body,
        grid=(N // NBLK,),
        in_specs=[pl.BlockSpec((NBLK, HF), lambda i: (i, 0)),
                  pl.BlockSpec((1, HF), fixed)],
        out_specs=pl.BlockSpec(memory_space=pltpu.SMEM),
        out_shape=jax.ShapeDtypeStruct((2,), jnp.float32),
    )(out512, gat_bias.reshape(1, HF))
    return pl.pallas_call(
        _k6b_body,
        grid=(N // NBLK,),
        in_specs=[
            pl.BlockSpec((NBLK, HF), lambda i: (i, 0)),
            pl.BlockSpec((1, HF), fixed),
            pl.BlockSpec(memory_space=pltpu.SMEM),
            pl.BlockSpec((1, HF), fixed),
            pl.BlockSpec((1, HF), fixed),
            pl.BlockSpec((HF, F), fixed),
            pl.BlockSpec((1, F), fixed),
            pl.BlockSpec((1, F), fixed),
            pl.BlockSpec((1, F), fixed),
            pl.BlockSpec((NBLK, F), lambda i: (i, 0)),
        ],
        out_specs=pl.BlockSpec((NBLK, F), lambda i: (i, 0)),
        out_shape=jax.ShapeDtypeStruct((N, F), jnp.float32),
    )(out512, gat_bias.reshape(1, HF), stats, gln_w.reshape(1, HF),
      gln_b.reshape(1, HF), postW, postb.reshape(1, F),
      post_ln_w.reshape(1, F), post_ln_b.reshape(1, F), h0)


def kernel(x, edge_index, edge_attr, embed_W, embed_b, embed_ln_w, embed_ln_b,
           Wl, bl, Wr, br, We, att, gat_bias, gln_w, gln_b,
           postW, postb, post_ln_w, post_ln_b):
    h0, xl, xr, xlb, xrb = _pre(x, embed_W, embed_b, embed_ln_w, embed_ln_b,
                                Wl, bl, Wr, br)

    loop = jnp.arange(N, dtype=jnp.int32)
    pad = jnp.zeros((EP - EHAT,), jnp.int32)
    srcp = jnp.concatenate([edge_index[0].astype(jnp.int32), loop, pad])
    dstp = jnp.concatenate([edge_index[1].astype(jnp.int32), loop, pad])
    eap = jnp.concatenate(
        [edge_attr, jnp.zeros((EP - E, D_EDGE), edge_attr.dtype)], axis=0)

    gsum = _sc_gsum(xlb, xrb, srcp, dstp)

    att_flat = att.reshape(1, HF)
    hsel = (jnp.arange(HF) // F)[None, :] == jnp.arange(8)[:, None]
    att8 = jnp.where(hsel, att_flat, 0.0).astype(jnp.float32)
    exT = _edge_alpha(gsum, eap, We, att8)

    den2 = _sc_den(dstp, exT)

    xlT = xl.reshape(N, H, F).transpose(1, 0, 2)
    src2 = srcp.reshape(EP // KCH, KCH)
    dst2 = dstp.reshape(EP // KCH, KCH)
    outT = _sc_msg(xlT, src2, dst2, exT, den2)

    out512 = outT[:, :N, :].transpose(1, 0, 2).reshape(N, HF)
    return _post(out512, gat_bias, gln_w, gln_b, postW, postb,
                 post_ln_w, post_ln_b, h0)
